# baseline (speedup 1.0000x reference)
import functools

import jax
import jax.numpy as jnp
from jax import lax
from jax.experimental import pallas as pl
from jax.experimental.pallas import tpu as pltpu
import jax.experimental.pallas.tpu_sc as plsc

N = 10000
E = 160000
C = 64
HC = 64
NRBF = 128
EC = 64
NL = 2
NSP = 100
CUT = 5.0
LCO = 9

NPAD = 10112
K = 128
NCH = E // K
NCORE = 2
NSUB = 16
NW = NCORE * NSUB
BLKN = 1000
BLKE = 2000
W = 128
F32 = jnp.float32

NBUF = 6


def _sc_gather2(tabS, tabR, sidx, ridx, npl):
    total = npl * NCH
    trips = (total + NW - 1) // NW
    trips2 = (trips + NBUF - 1) // NBUF
    mesh = plsc.VectorSubcoreMesh(core_axis_name="c", subcore_axis_name="s")

    @functools.partial(
        pl.kernel,
        out_type=(jax.ShapeDtypeStruct((npl * E, W), F32),
                  jax.ShapeDtypeStruct((npl * E, W), F32)),
        mesh=mesh,
        scratch_types=([pltpu.VMEM((K,), jnp.int32)] * NBUF
                       + [pltpu.VMEM((K, W), F32)] * NBUF
                       + [pltpu.SemaphoreType.DMA] * (3 * NBUF)),
    )
    def k(tabS_h, tabR_h, sidx_h, ridx_h, outS_h, outR_h, *scr):
        idx_v = scr[0:NBUF]
        rows_v = scr[NBUF:2 * NBUF]
        sem_i = scr[2 * NBUF:3 * NBUF]
        sem_g = scr[3 * NBUF:4 * NBUF]
        sem_w = scr[4 * NBUF:5 * NBUF]
        w = lax.axis_index("s") * NCORE + lax.axis_index("c")
        for tab_h, i_h, o_h in ((tabS_h, sidx_h, outS_h),
                                (tabR_h, ridx_h, outR_h)):
            for s in range(NBUF):
                g = w + NW * s

                @pl.when(g < total)
                def _(s=s, g=g, i_h=i_h):
                    pltpu.async_copy(i_h.at[pl.ds(g * K, K)], idx_v[s],
                                     sem_i[s])

            def body(jj, carry, tab_h=tab_h, i_h=i_h, o_h=o_h):
                j0 = jj * NBUF
                for s in range(NBUF):
                    j = j0 + s
                    g = w + NW * j

                    @pl.when(g < total)
                    def _(s=s, j=j, g=g):
                        pltpu.make_async_copy(i_h.at[pl.ds(g * K, K)],
                                              idx_v[s], sem_i[s]).wait()

                        @pl.when(j >= NBUF)
                        def __():
                            pltpu.make_async_copy(
                                rows_v[s], o_h.at[pl.ds(g * K, K)],
                                sem_w[s]).wait()
                        pltpu.async_copy(tab_h.at[idx_v[s]], rows_v[s],
                                         sem_g[s])
                for s in range(NBUF):
                    j = j0 + s
                    g = w + NW * j

                    @pl.when(g < total)
                    def _(s=s, j=j, g=g):
                        pltpu.make_async_copy(tab_h.at[idx_v[s]], rows_v[s],
                                              sem_g[s]).wait()
                        pltpu.async_copy(rows_v[s], o_h.at[pl.ds(g * K, K)],
                                         sem_w[s])
                        g2 = g + NW * NBUF

                        @pl.when(g2 < total)
                        def __(s=s, g2=g2):
                            pltpu.async_copy(i_h.at[pl.ds(g2 * K, K)],
                                             idx_v[s], sem_i[s])
                return carry
            lax.fori_loop(0, trips2, body, 0)
            for s in range(NBUF):
                @pl.when(w + NW * s < total)
                def _(s=s, o_h=o_h):
                    pltpu.make_async_copy(rows_v[s], o_h.at[pl.ds(0, K)],
                                          sem_w[s]).wait()

    return k(tabS, tabR, sidx, ridx)


def _sc_scatter(vals, ridx, zeros_h, npl):
    trips = (NCH + NW - 1) // NW
    NB = 3
    trips2 = (trips + NB - 1) // NB
    mesh = plsc.VectorSubcoreMesh(core_axis_name="c", subcore_axis_name="s")

    @functools.partial(
        pl.kernel,
        out_type=jax.ShapeDtypeStruct((2 * npl * NPAD, W), F32),
        mesh=mesh,
        scratch_types=([pltpu.VMEM((K,), jnp.int32)] * NB
                       + [pltpu.VMEM((K, W), F32)] * NB
                       + [pltpu.VMEM_SHARED((NPAD, W), F32)]
                       + [pltpu.SemaphoreType.DMA] * (2 * NB)),
    )
    def k(vals_h, ridx_h, zer_h, agg_h, *scr):
        idx_v = scr[0:NB]
        rowsA = scr[NB:2 * NB]
        tab = scr[2 * NB]
        sem_rA = scr[2 * NB + 1:3 * NB + 1]
        sem_aA = scr[3 * NB + 1:4 * NB + 1]
        cc = lax.axis_index("c")
        ss = lax.axis_index("s")
        w = ss * NCORE + cc
        zrows = NPAD // NSUB
        for p in range(npl):
            pltpu.sync_copy(zer_h, tab.at[pl.ds(ss * zrows, zrows)])
            plsc.subcore_barrier()

            def body(jj, carry, p=p):
                j0 = jj * NB
                for s in range(NB):
                    j = j0 + s
                    g = w + NW * j

                    @pl.when(g < NCH)
                    def _(s=s, j=j, g=g):
                        @pl.when(j >= NB)
                        def __():
                            pltpu.make_async_copy(
                                vals_h.at[pl.ds(0, K)], rowsA[s],
                                sem_aA[s]).wait()
                        base = g * K
                        pltpu.sync_copy(ridx_h.at[pl.ds(base, K)], idx_v[s])
                        pltpu.async_copy(
                            vals_h.at[pl.ds(p * E + base, K)],
                            rowsA[s], sem_rA[s])
                for s in range(NB):
                    j = j0 + s
                    g = w + NW * j

                    @pl.when(g < NCH)
                    def _(s=s, g=g):
                        base = g * K
                        pltpu.make_async_copy(
                            vals_h.at[pl.ds(p * E + base, K)],
                            rowsA[s], sem_rA[s]).wait()
                        pltpu.async_copy(rowsA[s], tab.at[idx_v[s]],
                                         sem_aA[s], add=True)
                return carry
            lax.fori_loop(0, trips2, body, 0)
            for s in range(NB):
                @pl.when(w + NW * s < NCH)
                def _(s=s):
                    pltpu.make_async_copy(vals_h.at[pl.ds(0, K)], rowsA[s],
                                          sem_aA[s]).wait()
            plsc.subcore_barrier()

            pltpu.sync_copy(
                tab.at[pl.ds(ss * zrows, zrows)],
                agg_h.at[pl.ds(cc * (npl * NPAD) + p * NPAD + ss * zrows,
                               zrows)])
            plsc.subcore_barrier()

    return k(vals, ridx, zeros_h)


def _norm_planes(xs, g_ref, b):
    eps = 1e-5
    x0 = xs[0]
    mu = jnp.mean(x0, axis=1, keepdims=True)
    xc = x0 - mu
    var = jnp.mean(xc * xc, axis=1, keepdims=True)
    y0 = xc * lax.rsqrt(var + eps) * g_ref[0:1]
    if b is not None:
        y0 = y0 + b
    ms1 = xs[1] * xs[1] + xs[2] * xs[2] + xs[3] * xs[3]
    i1 = lax.rsqrt(jnp.sum(ms1, axis=1, keepdims=True) * (1.0 / 192.0) + eps)
    ms2 = (xs[4] * xs[4] + xs[5] * xs[5] + xs[6] * xs[6]
           + xs[7] * xs[7] + xs[8] * xs[8])
    i2 = lax.rsqrt(jnp.sum(ms2, axis=1, keepdims=True) * (1.0 / 320.0) + eps)
    out = [y0]
    for l in range(1, 4):
        out.append(xs[l] * i1 * g_ref[1:2])
    for l in range(4, 9):
        out.append(xs[l] * i2 * g_ref[2:3])
    return out


def _pack_pairs(ref, planes):
    z = jnp.zeros_like(planes[0])
    for p in range(5):
        hi = planes[2 * p + 1] if 2 * p + 1 < 9 else z
        ref[p] = jnp.concatenate([planes[2 * p], hi], axis=1)


def _prep_kernel(sp_ref, csd_ref, sphP_ref, sendP_ref, recvP_ref,
                 Wsend_ref, Wrecv_ref, scal_ref, Ns_ref, Nr_ref):
    sp = sp_ref[...]
    lanes = lax.broadcasted_iota(jnp.int32, (BLKN, 128), 1)
    oh = (lanes == sp).astype(F32)
    dot = functools.partial(jnp.dot, preferred_element_type=F32)
    scal_ref[...] = dot(oh, sphP_ref[...]) + csd_ref[...]
    ssv = dot(oh, sendP_ref[...])
    rrv = dot(oh, recvP_ref[...])
    z = jnp.zeros((BLKN, C), F32)
    for ref, vec, Wref in ((Ns_ref, ssv, Wsend_ref), (Nr_ref, rrv, Wrecv_ref)):
        p0 = dot(vec, Wref[0])
        p1 = dot(vec, Wref[1])
        p2 = dot(vec, Wref[2])
        ref[0] = jnp.concatenate([p0, p1], axis=1)
        ref[1] = jnp.concatenate([p2, z], axis=1)


def _edge0_kernel(ev_ref, eS_ref, eR_ref, Wd1r_ref, Wd2_ref, Wd3r_ref,
                  Wr1r_ref, Wr2_ref, m0_ref, gates_ref, envb_ref):
    dot = functools.partial(jnp.dot, preferred_element_type=F32)
    ev = ev_ref[...]
    d2 = jnp.sum(ev * ev, axis=1, keepdims=True) + 1e-12
    d = jnp.sqrt(d2)
    u = d * (1.0 / CUT)
    u2 = u * u
    u5 = u2 * u2 * u
    env = 1.0 - 21.0 * u5 + 35.0 * u5 * u - 15.0 * u5 * u2
    env = jnp.where(u < 1.0, env, 0.0)
    mu = (lax.broadcasted_iota(jnp.int32, (1, NRBF), 1).astype(F32)
          * (CUT / (NRBF - 1)))
    t = d - mu
    sigma = 2.0 * CUT / NRBF
    rbf = jnp.exp(t * t * (-1.0 / (2.0 * sigma * sigma)))
    comb = eS_ref[...] + eR_ref[...]
    e_d = comb[0, :, 0:64]
    e_r0 = comb[0, :, 64:128]
    e_r1 = comb[1, :, 0:64]
    eh = jax.nn.silu(dot(rbf, Wd1r_ref[...]) + e_d)
    eh = jax.nn.silu(dot(eh, Wd2_ref[...]))
    z = jnp.zeros((BLKE, C), F32)
    m0a = dot(eh, Wd3r_ref[0]) * env
    m0b = dot(eh, Wd3r_ref[1]) * env
    m0c = dot(eh, Wd3r_ref[2]) * env
    m0_ref[0] = jnp.concatenate([m0a, m0b], axis=1)
    m0_ref[1] = jnp.concatenate([m0c, z], axis=1)
    for m, e_r in ((0, e_r0), (1, e_r1)):
        gm = jax.nn.silu(dot(rbf, Wr1r_ref[m]) + e_r)
        gates_ref[m] = jax.nn.silu(dot(gm, Wr2_ref[m]))
    envb_ref[...] = jnp.broadcast_to(env, (BLKE, 64))


def _n1_kernel(scal_ref, agg_ref, g_ref, b_ref, Wmi_ref, x_ref, h_ref):
    pair0 = agg_ref[0, 0] + agg_ref[1, 0]
    pair1 = agg_ref[0, 1] + agg_ref[1, 1]
    a0 = pair0[:, 0:64] * 0.2
    a1 = pair0[:, 64:128] * 0.2
    a2 = pair1[:, 0:64] * 0.2
    x0 = scal_ref[...] + a0
    z = jnp.zeros_like(x0)
    xs = [x0, z, a1, z, z, z, a2, z, z]
    for l in range(9):
        x_ref[l] = xs[l]
    hs = _norm_planes(xs, g_ref, b_ref[...])
    dot = functools.partial(jnp.dot, preferred_element_type=F32)
    Wmi = Wmi_ref[...]
    _pack_pairs(h_ref, [dot(p, Wmi) for p in hs])


def _edge_kernel(mS_ref, mR_ref, gate_ref, envb_ref, Wmo2_ref,
                 o_ref):
    dot = functools.partial(jnp.dot, preferred_element_type=F32)
    gate = gate_ref[0]
    gate2 = jnp.concatenate([gate, gate], axis=1)
    env = envb_ref[...]
    env2 = jnp.concatenate([env, env], axis=1)
    Wmo2 = Wmo2_ref[...]
    hm = [(mS_ref[p] + mR_ref[p]) * gate2 for p in range(5)]
    s = jax.nn.sigmoid(hm[0][:, 0:64])
    s2 = jnp.concatenate([s, s], axis=1)
    for p in range(5):
        o_ref[p] = dot(hm[p] * s2, Wmo2) * env2


def _make_n2_kernel(want_x):
    def _n2(x_ref, agg_ref, g2_ref, b2_ref, Wa1_ref, Wa2_ref,
            gn_ref, bn_ref, *rest):
        if want_x:
            Wminext_ref = rest[0]
            outs = rest[1:]
        else:
            outs = rest
        dot = functools.partial(jnp.dot, preferred_element_type=F32)
        xs = []
        for l in range(9):
            p, hh = divmod(l, 2)
            part = (agg_ref[0, p, :, 64 * hh:64 * hh + 64]
                    + agg_ref[1, p, :, 64 * hh:64 * hh + 64])
            xs.append(x_ref[l] + part)
        h2 = _norm_planes(xs, g2_ref, b2_ref[...])
        Wa1 = Wa1_ref[...]
        Wa2 = Wa2_ref[...]
        ha = [dot(h2[l], Wa1) for l in range(9)]
        s = jax.nn.sigmoid(ha[0])
        xs = [xs[l] + dot(ha[l] * s, Wa2) for l in range(9)]
        hn = _norm_planes(xs, gn_ref, bn_ref[...])
        if want_x:
            for l in range(9):
                outs[0][l] = xs[l]
            Wminext = Wminext_ref[...]
            _pack_pairs(outs[1], [dot(p, Wminext) for p in hn])
        else:
            for l in range(9):
                outs[0][l] = hn[l]
    return _n2


def _full(shape):
    nd = len(shape)
    return pl.BlockSpec(shape, lambda i, _n=nd: (0,) * _n)


def kernel(edge_vectors, csd_mixed_emb, sphere_emb, send_emb, recv_emb,
           Wd1, Wd2, Wd3, g1, b1, Wr1, Wr2, Wmi, Wmo, g2, b2, Wa1, Wa2,
           gf, bf, node_species, senders, receivers, n_node):
    f32 = F32
    snd = senders.astype(jnp.int32)
    rcv = receivers.astype(jnp.int32)
    sp2d = node_species.astype(jnp.int32).reshape(N, 1)
    ev4 = jnp.pad(edge_vectors.astype(f32), ((0, 0), (0, 1)))
    zeros_h = jnp.zeros((NPAD // NSUB, W), f32)

    pl2 = jnp.arange(2, dtype=jnp.int32)[:, None] * N
    pl5 = jnp.arange(5, dtype=jnp.int32)[:, None] * N
    sidx2 = (pl2 + snd[None, :]).reshape(-1)
    ridx2 = (pl2 + rcv[None, :]).reshape(-1)
    sidx5 = (pl5 + snd[None, :]).reshape(-1)
    ridx5 = (pl5 + rcv[None, :]).reshape(-1)

    sphP = jnp.pad(sphere_emb.astype(f32), ((0, 128 - NSP), (0, 0)))
    sendP = jnp.pad(send_emb.astype(f32), ((0, 128 - NSP), (0, 0)))
    recvP = jnp.pad(recv_emb.astype(f32), ((0, 128 - NSP), (0, 0)))
    Wsend = jnp.stack([Wd1[NRBF:NRBF + EC], Wr1[0, NRBF:NRBF + EC],
                       Wr1[1, NRBF:NRBF + EC]])
    Wrecv = jnp.stack([Wd1[NRBF + EC:], Wr1[0, NRBF + EC:],
                       Wr1[1, NRBF + EC:]])
    Wd1r = Wd1[:NRBF]
    Wr1r = Wr1[:, :NRBF]
    Wd3r = Wd3.reshape(EC, 3, C).transpose(1, 0, 2)
    zz = jnp.zeros((NL, W, W), f32)
    Wmo2 = zz.at[:, :64, :64].set(Wmo).at[:, 64:, 64:].set(Wmo)

    gn = N // BLKN
    scal, NsP, NrP = pl.pallas_call(
        _prep_kernel,
        out_shape=(jax.ShapeDtypeStruct((N, C), f32),
                   jax.ShapeDtypeStruct((2, N, W), f32),
                   jax.ShapeDtypeStruct((2, N, W), f32)),
        grid=(gn,),
        in_specs=[pl.BlockSpec((BLKN, 1), lambda i: (i, 0)),
                  _full((1, C)), _full((128, C)), _full((128, C)),
                  _full((128, C)), _full((3, EC, C)), _full((3, EC, C))],
        out_specs=[pl.BlockSpec((BLKN, C), lambda i: (i, 0)),
                   pl.BlockSpec((2, BLKN, W), lambda i: (0, i, 0)),
                   pl.BlockSpec((2, BLKN, W), lambda i: (0, i, 0))],
    )(sp2d, csd_mixed_emb, sphP, sendP, recvP, Wsend, Wrecv)

    eSf, eRf = _sc_gather2(NsP.reshape(2 * N, W), NrP.reshape(2 * N, W),
                           sidx2, ridx2, 2)

    ge = E // BLKE
    m0p, gates, envb = pl.pallas_call(
        _edge0_kernel,
        out_shape=(jax.ShapeDtypeStruct((2, E, W), f32),
                   jax.ShapeDtypeStruct((2, E, C), f32),
                   jax.ShapeDtypeStruct((E, C), f32)),
        grid=(ge,),
        in_specs=[pl.BlockSpec((BLKE, 4), lambda i: (i, 0)),
                  pl.BlockSpec((2, BLKE, W), lambda i: (0, i, 0)),
                  pl.BlockSpec((2, BLKE, W), lambda i: (0, i, 0)),
                  _full((NRBF, EC)), _full((EC, EC)), _full((3, EC, C)),
                  _full((2, NRBF, EC)), _full((2, EC, HC))],
        out_specs=[pl.BlockSpec((2, BLKE, W), lambda i: (0, i, 0)),
                   pl.BlockSpec((2, BLKE, C), lambda i: (0, i, 0)),
                   pl.BlockSpec((BLKE, C), lambda i: (i, 0))],
    )(ev4, eSf.reshape(2, E, W), eRf.reshape(2, E, W),
      Wd1r, Wd2, Wd3r, Wr1r, Wr2)

    agg0 = _sc_scatter(m0p.reshape(2 * E, W), rcv, zeros_h, 2)

    x, h = pl.pallas_call(
        _n1_kernel,
        out_shape=(jax.ShapeDtypeStruct((9, N, C), f32),
                   jax.ShapeDtypeStruct((5, N, W), f32)),
        grid=(gn,),
        in_specs=[pl.BlockSpec((BLKN, C), lambda i: (i, 0)),
                  pl.BlockSpec((2, 2, BLKN, W), lambda i: (0, 0, i, 0)),
                  _full((3, C)), _full((1, C)), _full((C, HC))],
        out_specs=[pl.BlockSpec((9, BLKN, C), lambda i: (0, i, 0)),
                   pl.BlockSpec((5, BLKN, W), lambda i: (0, i, 0))],
    )(scal, agg0.reshape(2, 2, NPAD, W), g1[0], b1[0].reshape(1, C), Wmi[0])

    for i in range(NL):
        mS, mR = _sc_gather2(h.reshape(5 * N, W), h.reshape(5 * N, W),
                             sidx5, ridx5, 5)
        outp = pl.pallas_call(
            _edge_kernel,
            out_shape=jax.ShapeDtypeStruct((5, E, W), f32),
            grid=(ge,),
            in_specs=[pl.BlockSpec((5, BLKE, W), lambda j: (0, j, 0)),
                      pl.BlockSpec((5, BLKE, W), lambda j: (0, j, 0)),
                      pl.BlockSpec((1, BLKE, C), lambda j, _i=i: (_i, j, 0)),
                      pl.BlockSpec((BLKE, C), lambda j: (j, 0)),
                      _full((W, W))],
            out_specs=pl.BlockSpec((5, BLKE, W), lambda j: (0, j, 0)),
        )(mS.reshape(5, E, W), mR.reshape(5, E, W), gates, envb,
          Wmo2[i])

        aggp = _sc_scatter(outp.reshape(5 * E, W), rcv, zeros_h, 5)

        last = i == NL - 1
        gnx = gf if last else g1[i + 1]
        bnx = bf if last else b1[i + 1]
        if last:
            outs = (jax.ShapeDtypeStruct((9, N, C), f32),)
            out_specs = [pl.BlockSpec((9, BLKN, C), lambda j: (0, j, 0))]
        else:
            outs = (jax.ShapeDtypeStruct((9, N, C), f32),
                    jax.ShapeDtypeStruct((5, N, W), f32))
            out_specs = [pl.BlockSpec((9, BLKN, C), lambda j: (0, j, 0)),
                         pl.BlockSpec((5, BLKN, W), lambda j: (0, j, 0))]
        in_specs = [pl.BlockSpec((9, BLKN, C), lambda j: (0, j, 0)),
                    pl.BlockSpec((2, 5, BLKN, W), lambda j: (0, 0, j, 0)),
                    _full((3, C)), _full((1, C)),
                    _full((C, HC)), _full((HC, C)),
                    _full((3, C)), _full((1, C))]
        args = [x, aggp.reshape(2, 5, NPAD, W), g2[i], b2[i].reshape(1, C),
                Wa1[i], Wa2[i], gnx, bnx.reshape(1, C)]
        if not last:
            in_specs.append(_full((C, HC)))
            args.append(Wmi[i + 1])
        res = pl.pallas_call(
            _make_n2_kernel(want_x=not last),
            out_shape=outs,
            grid=(gn,),
            in_specs=in_specs,
            out_specs=out_specs,
        )(*args)
        if last:
            hfin = res[0]
        else:
            x, h = res

    return jnp.transpose(hfin, (1, 0, 2))


# speedup vs baseline: 1.0021x; 1.0021x over previous
import functools

import jax
import jax.numpy as jnp
from jax import lax
from jax.experimental import pallas as pl
from jax.experimental.pallas import tpu as pltpu
import jax.experimental.pallas.tpu_sc as plsc

N = 10000
E = 160000
C = 64
HC = 64
NRBF = 128
EC = 64
NL = 2
NSP = 100
CUT = 5.0
LCO = 9

NPAD = 10112
K = 128
NCH = E // K
NCORE = 2
NSUB = 16
NW = NCORE * NSUB
BLKN = 1000
BLKE = 2000
W = 128
F32 = jnp.float32

NBUF = 6


def _sc_gather2(tabS, tabR, sidx, ridx, npl):
    total = npl * NCH
    trips = (total + NW - 1) // NW
    trips2 = (trips + NBUF - 1) // NBUF
    mesh = plsc.VectorSubcoreMesh(core_axis_name="c", subcore_axis_name="s")

    @functools.partial(
        pl.kernel,
        out_type=(jax.ShapeDtypeStruct((npl * E, W), F32),
                  jax.ShapeDtypeStruct((npl * E, W), F32)),
        mesh=mesh,
        scratch_types=([pltpu.VMEM((K,), jnp.int32)] * NBUF
                       + [pltpu.VMEM((K, W), F32)] * NBUF
                       + [pltpu.SemaphoreType.DMA] * (3 * NBUF)),
    )
    def k(tabS_h, tabR_h, sidx_h, ridx_h, outS_h, outR_h, *scr):
        idx_v = scr[0:NBUF]
        rows_v = scr[NBUF:2 * NBUF]
        sem_i = scr[2 * NBUF:3 * NBUF]
        sem_g = scr[3 * NBUF:4 * NBUF]
        sem_w = scr[4 * NBUF:5 * NBUF]
        w = lax.axis_index("s") * NCORE + lax.axis_index("c")
        for tab_h, i_h, o_h in ((tabS_h, sidx_h, outS_h),
                                (tabR_h, ridx_h, outR_h)):
            for s in range(NBUF):
                g = w + NW * s

                @pl.when(g < total)
                def _(s=s, g=g, i_h=i_h):
                    pltpu.async_copy(i_h.at[pl.ds(g * K, K)], idx_v[s],
                                     sem_i[s])

            def body(jj, carry, tab_h=tab_h, i_h=i_h, o_h=o_h):
                j0 = jj * NBUF
                for s in range(NBUF):
                    j = j0 + s
                    g = w + NW * j

                    @pl.when(g < total)
                    def _(s=s, j=j, g=g):
                        pltpu.make_async_copy(i_h.at[pl.ds(g * K, K)],
                                              idx_v[s], sem_i[s]).wait()

                        @pl.when(j >= NBUF)
                        def __():
                            pltpu.make_async_copy(
                                rows_v[s], o_h.at[pl.ds(g * K, K)],
                                sem_w[s]).wait()
                        pltpu.async_copy(tab_h.at[idx_v[s]], rows_v[s],
                                         sem_g[s])
                for s in range(NBUF):
                    j = j0 + s
                    g = w + NW * j

                    @pl.when(g < total)
                    def _(s=s, j=j, g=g):
                        pltpu.make_async_copy(tab_h.at[idx_v[s]], rows_v[s],
                                              sem_g[s]).wait()
                        pltpu.async_copy(rows_v[s], o_h.at[pl.ds(g * K, K)],
                                         sem_w[s])
                        g2 = g + NW * NBUF

                        @pl.when(g2 < total)
                        def __(s=s, g2=g2):
                            pltpu.async_copy(i_h.at[pl.ds(g2 * K, K)],
                                             idx_v[s], sem_i[s])
                return carry
            lax.fori_loop(0, trips2, body, 0)
            for s in range(NBUF):
                @pl.when(w + NW * s < total)
                def _(s=s, o_h=o_h):
                    pltpu.make_async_copy(rows_v[s], o_h.at[pl.ds(0, K)],
                                          sem_w[s]).wait()

    return k(tabS, tabR, sidx, ridx)


def _sc_scatter(vals, ridx, zeros_h, npl):
    trips = (NCH + NW - 1) // NW
    NB = 3
    trips2 = (trips + NB - 1) // NB
    mesh = plsc.VectorSubcoreMesh(core_axis_name="c", subcore_axis_name="s")

    @functools.partial(
        pl.kernel,
        out_type=jax.ShapeDtypeStruct((2 * npl * NPAD, W), F32),
        mesh=mesh,
        scratch_types=([pltpu.VMEM((K,), jnp.int32)] * NB
                       + [pltpu.VMEM((K, W), F32)] * NB
                       + [pltpu.VMEM_SHARED((NPAD, W), F32)]
                       + [pltpu.SemaphoreType.DMA] * (2 * NB)),
    )
    def k(vals_h, ridx_h, zer_h, agg_h, *scr):
        idx_v = scr[0:NB]
        rowsA = scr[NB:2 * NB]
        tab = scr[2 * NB]
        sem_rA = scr[2 * NB + 1:3 * NB + 1]
        sem_aA = scr[3 * NB + 1:4 * NB + 1]
        cc = lax.axis_index("c")
        ss = lax.axis_index("s")
        w = ss * NCORE + cc
        zrows = NPAD // NSUB
        for p in range(npl):
            pltpu.sync_copy(zer_h, tab.at[pl.ds(ss * zrows, zrows)])
            plsc.subcore_barrier()

            def body(jj, carry, p=p):
                j0 = jj * NB
                for s in range(NB):
                    j = j0 + s
                    g = w + NW * j

                    @pl.when(g < NCH)
                    def _(s=s, j=j, g=g):
                        @pl.when(j >= NB)
                        def __():
                            pltpu.make_async_copy(
                                vals_h.at[pl.ds(0, K)], rowsA[s],
                                sem_aA[s]).wait()
                        base = g * K
                        pltpu.sync_copy(ridx_h.at[pl.ds(base, K)], idx_v[s])
                        pltpu.async_copy(
                            vals_h.at[pl.ds(p * E + base, K)],
                            rowsA[s], sem_rA[s])
                for s in range(NB):
                    j = j0 + s
                    g = w + NW * j

                    @pl.when(g < NCH)
                    def _(s=s, g=g):
                        base = g * K
                        pltpu.make_async_copy(
                            vals_h.at[pl.ds(p * E + base, K)],
                            rowsA[s], sem_rA[s]).wait()
                        pltpu.async_copy(rowsA[s], tab.at[idx_v[s]],
                                         sem_aA[s], add=True)
                return carry
            lax.fori_loop(0, trips2, body, 0)
            for s in range(NB):
                @pl.when(w + NW * s < NCH)
                def _(s=s):
                    pltpu.make_async_copy(vals_h.at[pl.ds(0, K)], rowsA[s],
                                          sem_aA[s]).wait()
            plsc.subcore_barrier()

            pltpu.sync_copy(
                tab.at[pl.ds(ss * zrows, zrows)],
                agg_h.at[pl.ds(cc * (npl * NPAD) + p * NPAD + ss * zrows,
                               zrows)])
            plsc.subcore_barrier()

    return k(vals, ridx, zeros_h)


def _norm_planes(xs, g_ref, b):
    eps = 1e-5
    x0 = xs[0]
    mu = jnp.mean(x0, axis=1, keepdims=True)
    xc = x0 - mu
    var = jnp.mean(xc * xc, axis=1, keepdims=True)
    y0 = xc * lax.rsqrt(var + eps) * g_ref[0:1]
    if b is not None:
        y0 = y0 + b
    ms1 = xs[1] * xs[1] + xs[2] * xs[2] + xs[3] * xs[3]
    i1 = lax.rsqrt(jnp.sum(ms1, axis=1, keepdims=True) * (1.0 / 192.0) + eps)
    ms2 = (xs[4] * xs[4] + xs[5] * xs[5] + xs[6] * xs[6]
           + xs[7] * xs[7] + xs[8] * xs[8])
    i2 = lax.rsqrt(jnp.sum(ms2, axis=1, keepdims=True) * (1.0 / 320.0) + eps)
    out = [y0]
    for l in range(1, 4):
        out.append(xs[l] * i1 * g_ref[1:2])
    for l in range(4, 9):
        out.append(xs[l] * i2 * g_ref[2:3])
    return out


def _pack_pairs(ref, planes):
    z = jnp.zeros_like(planes[0])
    for p in range(5):
        hi = planes[2 * p + 1] if 2 * p + 1 < 9 else z
        ref[p] = jnp.concatenate([planes[2 * p], hi], axis=1)


def _prep_kernel(sp_ref, csd_ref, sphP_ref, sendP_ref, recvP_ref,
                 Wsend_ref, Wrecv_ref, scal_ref, Ns_ref, Nr_ref):
    sp = sp_ref[...]
    lanes = lax.broadcasted_iota(jnp.int32, (BLKN, 128), 1)
    oh = (lanes == sp).astype(F32)
    dot = functools.partial(jnp.dot, preferred_element_type=F32)
    scal_ref[...] = dot(oh, sphP_ref[...]) + csd_ref[...]
    ssv = dot(oh, sendP_ref[...])
    rrv = dot(oh, recvP_ref[...])
    z = jnp.zeros((BLKN, C), F32)
    for ref, vec, Wref in ((Ns_ref, ssv, Wsend_ref), (Nr_ref, rrv, Wrecv_ref)):
        p0 = dot(vec, Wref[0])
        p1 = dot(vec, Wref[1])
        p2 = dot(vec, Wref[2])
        ref[0] = jnp.concatenate([p0, p1], axis=1)
        ref[1] = jnp.concatenate([p2, z], axis=1)


def _edge0_kernel(ev_ref, eS_ref, eR_ref, Wd1r_ref, Wd2_ref, Wd3r_ref,
                  Wr1r_ref, Wr2_ref, m0_ref, gates_ref, envb_ref):
    dot = functools.partial(jnp.dot, preferred_element_type=F32)
    ev = ev_ref[...]
    d2 = jnp.sum(ev * ev, axis=1, keepdims=True) + 1e-12
    d = jnp.sqrt(d2)
    u = d * (1.0 / CUT)
    u2 = u * u
    u5 = u2 * u2 * u
    env = 1.0 - 21.0 * u5 + 35.0 * u5 * u - 15.0 * u5 * u2
    env = jnp.where(u < 1.0, env, 0.0)
    mu = (lax.broadcasted_iota(jnp.int32, (1, NRBF), 1).astype(F32)
          * (CUT / (NRBF - 1)))
    t = d - mu
    sigma = 2.0 * CUT / NRBF
    rbf = jnp.exp(t * t * (-1.0 / (2.0 * sigma * sigma)))
    comb = eS_ref[...] + eR_ref[...]
    e_d = comb[0, :, 0:64]
    e_r0 = comb[0, :, 64:128]
    e_r1 = comb[1, :, 0:64]
    eh = jax.nn.silu(dot(rbf, Wd1r_ref[...]) + e_d)
    eh = jax.nn.silu(dot(eh, Wd2_ref[...]))
    z = jnp.zeros((BLKE, C), F32)
    m0a = dot(eh, Wd3r_ref[0]) * env
    m0b = dot(eh, Wd3r_ref[1]) * env
    m0c = dot(eh, Wd3r_ref[2]) * env
    m0_ref[0] = jnp.concatenate([m0a, m0b], axis=1)
    m0_ref[1] = jnp.concatenate([m0c, z], axis=1)
    for m, e_r in ((0, e_r0), (1, e_r1)):
        gm = jax.nn.silu(dot(rbf, Wr1r_ref[m]) + e_r)
        gates_ref[m] = jax.nn.silu(dot(gm, Wr2_ref[m]))
    envb_ref[...] = jnp.broadcast_to(env, (BLKE, 64))


def _n1_kernel(scal_ref, agg_ref, g_ref, b_ref, x_ref, h_ref):
    pair0 = agg_ref[0, 0] + agg_ref[1, 0]
    pair1 = agg_ref[0, 1] + agg_ref[1, 1]
    a0 = pair0[:, 0:64] * 0.2
    a1 = pair0[:, 64:128] * 0.2
    a2 = pair1[:, 0:64] * 0.2
    x0 = scal_ref[...] + a0
    z = jnp.zeros_like(x0)
    xs = [x0, z, a1, z, z, z, a2, z, z]
    for l in range(9):
        x_ref[l] = xs[l]
    hs = _norm_planes(xs, g_ref, b_ref[...])
    _pack_pairs(h_ref, hs)


def _edge_kernel(mS_ref, mR_ref, gate_ref, envb_ref, Wmi2_ref, Wmo2_ref,
                 o_ref):
    dot = functools.partial(jnp.dot, preferred_element_type=F32)
    gate = gate_ref[0]
    gate2 = jnp.concatenate([gate, gate], axis=1)
    env = envb_ref[...]
    env2 = jnp.concatenate([env, env], axis=1)
    Wmi2 = Wmi2_ref[...]
    Wmo2 = Wmo2_ref[...]
    hm = [dot(mS_ref[p] + mR_ref[p], Wmi2) * gate2 for p in range(5)]
    s = jax.nn.sigmoid(hm[0][:, 0:64])
    s2 = jnp.concatenate([s, s], axis=1)
    for p in range(5):
        o_ref[p] = dot(hm[p] * s2, Wmo2) * env2


def _make_n2_kernel(want_x):
    def _n2(x_ref, agg_ref, g2_ref, b2_ref, Wa1_ref, Wa2_ref,
            gn_ref, bn_ref, *outs):
        dot = functools.partial(jnp.dot, preferred_element_type=F32)
        xs = []
        for l in range(9):
            p, hh = divmod(l, 2)
            part = (agg_ref[0, p, :, 64 * hh:64 * hh + 64]
                    + agg_ref[1, p, :, 64 * hh:64 * hh + 64])
            xs.append(x_ref[l] + part)
        h2 = _norm_planes(xs, g2_ref, b2_ref[...])
        Wa1 = Wa1_ref[...]
        Wa2 = Wa2_ref[...]
        ha = [dot(h2[l], Wa1) for l in range(9)]
        s = jax.nn.sigmoid(ha[0])
        xs = [xs[l] + dot(ha[l] * s, Wa2) for l in range(9)]
        hn = _norm_planes(xs, gn_ref, bn_ref[...])
        if want_x:
            for l in range(9):
                outs[0][l] = xs[l]
            _pack_pairs(outs[1], hn)
        else:
            for l in range(9):
                outs[0][l] = hn[l]
    return _n2


def _full(shape):
    nd = len(shape)
    return pl.BlockSpec(shape, lambda i, _n=nd: (0,) * _n)


def kernel(edge_vectors, csd_mixed_emb, sphere_emb, send_emb, recv_emb,
           Wd1, Wd2, Wd3, g1, b1, Wr1, Wr2, Wmi, Wmo, g2, b2, Wa1, Wa2,
           gf, bf, node_species, senders, receivers, n_node):
    f32 = F32
    snd = senders.astype(jnp.int32)
    rcv = receivers.astype(jnp.int32)
    sp2d = node_species.astype(jnp.int32).reshape(N, 1)
    ev4 = jnp.pad(edge_vectors.astype(f32), ((0, 0), (0, 1)))
    zeros_h = jnp.zeros((NPAD // NSUB, W), f32)

    pl2 = jnp.arange(2, dtype=jnp.int32)[:, None] * N
    pl5 = jnp.arange(5, dtype=jnp.int32)[:, None] * N
    sidx2 = (pl2 + snd[None, :]).reshape(-1)
    ridx2 = (pl2 + rcv[None, :]).reshape(-1)
    sidx5 = (pl5 + snd[None, :]).reshape(-1)
    ridx5 = (pl5 + rcv[None, :]).reshape(-1)

    sphP = jnp.pad(sphere_emb.astype(f32), ((0, 128 - NSP), (0, 0)))
    sendP = jnp.pad(send_emb.astype(f32), ((0, 128 - NSP), (0, 0)))
    recvP = jnp.pad(recv_emb.astype(f32), ((0, 128 - NSP), (0, 0)))
    Wsend = jnp.stack([Wd1[NRBF:NRBF + EC], Wr1[0, NRBF:NRBF + EC],
                       Wr1[1, NRBF:NRBF + EC]])
    Wrecv = jnp.stack([Wd1[NRBF + EC:], Wr1[0, NRBF + EC:],
                       Wr1[1, NRBF + EC:]])
    Wd1r = Wd1[:NRBF]
    Wr1r = Wr1[:, :NRBF]
    Wd3r = Wd3.reshape(EC, 3, C).transpose(1, 0, 2)
    zz = jnp.zeros((NL, W, W), f32)
    Wmi2 = zz.at[:, :64, :64].set(Wmi).at[:, 64:, 64:].set(Wmi)
    Wmo2 = zz.at[:, :64, :64].set(Wmo).at[:, 64:, 64:].set(Wmo)

    gn = N // BLKN
    scal, NsP, NrP = pl.pallas_call(
        _prep_kernel,
        out_shape=(jax.ShapeDtypeStruct((N, C), f32),
                   jax.ShapeDtypeStruct((2, N, W), f32),
                   jax.ShapeDtypeStruct((2, N, W), f32)),
        grid=(gn,),
        in_specs=[pl.BlockSpec((BLKN, 1), lambda i: (i, 0)),
                  _full((1, C)), _full((128, C)), _full((128, C)),
                  _full((128, C)), _full((3, EC, C)), _full((3, EC, C))],
        out_specs=[pl.BlockSpec((BLKN, C), lambda i: (i, 0)),
                   pl.BlockSpec((2, BLKN, W), lambda i: (0, i, 0)),
                   pl.BlockSpec((2, BLKN, W), lambda i: (0, i, 0))],
    )(sp2d, csd_mixed_emb, sphP, sendP, recvP, Wsend, Wrecv)

    eSf, eRf = _sc_gather2(NsP.reshape(2 * N, W), NrP.reshape(2 * N, W),
                           sidx2, ridx2, 2)

    ge = E // BLKE
    m0p, gates, envb = pl.pallas_call(
        _edge0_kernel,
        out_shape=(jax.ShapeDtypeStruct((2, E, W), f32),
                   jax.ShapeDtypeStruct((2, E, C), f32),
                   jax.ShapeDtypeStruct((E, C), f32)),
        grid=(ge,),
        in_specs=[pl.BlockSpec((BLKE, 4), lambda i: (i, 0)),
                  pl.BlockSpec((2, BLKE, W), lambda i: (0, i, 0)),
                  pl.BlockSpec((2, BLKE, W), lambda i: (0, i, 0)),
                  _full((NRBF, EC)), _full((EC, EC)), _full((3, EC, C)),
                  _full((2, NRBF, EC)), _full((2, EC, HC))],
        out_specs=[pl.BlockSpec((2, BLKE, W), lambda i: (0, i, 0)),
                   pl.BlockSpec((2, BLKE, C), lambda i: (0, i, 0)),
                   pl.BlockSpec((BLKE, C), lambda i: (i, 0))],
    )(ev4, eSf.reshape(2, E, W), eRf.reshape(2, E, W),
      Wd1r, Wd2, Wd3r, Wr1r, Wr2)

    agg0 = _sc_scatter(m0p.reshape(2 * E, W), rcv, zeros_h, 2)

    x, h = pl.pallas_call(
        _n1_kernel,
        out_shape=(jax.ShapeDtypeStruct((9, N, C), f32),
                   jax.ShapeDtypeStruct((5, N, W), f32)),
        grid=(gn,),
        in_specs=[pl.BlockSpec((BLKN, C), lambda i: (i, 0)),
                  pl.BlockSpec((2, 2, BLKN, W), lambda i: (0, 0, i, 0)),
                  _full((3, C)), _full((1, C))],
        out_specs=[pl.BlockSpec((9, BLKN, C), lambda i: (0, i, 0)),
                   pl.BlockSpec((5, BLKN, W), lambda i: (0, i, 0))],
    )(scal, agg0.reshape(2, 2, NPAD, W), g1[0], b1[0].reshape(1, C))

    for i in range(NL):
        mS, mR = _sc_gather2(h.reshape(5 * N, W), h.reshape(5 * N, W),
                             sidx5, ridx5, 5)
        outp = pl.pallas_call(
            _edge_kernel,
            out_shape=jax.ShapeDtypeStruct((5, E, W), f32),
            grid=(ge,),
            in_specs=[pl.BlockSpec((5, BLKE, W), lambda j: (0, j, 0)),
                      pl.BlockSpec((5, BLKE, W), lambda j: (0, j, 0)),
                      pl.BlockSpec((1, BLKE, C), lambda j, _i=i: (_i, j, 0)),
                      pl.BlockSpec((BLKE, C), lambda j: (j, 0)),
                      _full((W, W)), _full((W, W))],
            out_specs=pl.BlockSpec((5, BLKE, W), lambda j: (0, j, 0)),
        )(mS.reshape(5, E, W), mR.reshape(5, E, W), gates, envb,
          Wmi2[i], Wmo2[i])

        aggp = _sc_scatter(outp.reshape(5 * E, W), rcv, zeros_h, 5)

        last = i == NL - 1
        gnx = gf if last else g1[i + 1]
        bnx = bf if last else b1[i + 1]
        if last:
            outs = (jax.ShapeDtypeStruct((9, N, C), f32),)
            out_specs = [pl.BlockSpec((9, BLKN, C), lambda j: (0, j, 0))]
        else:
            outs = (jax.ShapeDtypeStruct((9, N, C), f32),
                    jax.ShapeDtypeStruct((5, N, W), f32))
            out_specs = [pl.BlockSpec((9, BLKN, C), lambda j: (0, j, 0)),
                         pl.BlockSpec((5, BLKN, W), lambda j: (0, j, 0))]
        res = pl.pallas_call(
            _make_n2_kernel(want_x=not last),
            out_shape=outs,
            grid=(gn,),
            in_specs=[pl.BlockSpec((9, BLKN, C), lambda j: (0, j, 0)),
                      pl.BlockSpec((2, 5, BLKN, W), lambda j: (0, 0, j, 0)),
                      _full((3, C)), _full((1, C)),
                      _full((C, HC)), _full((HC, C)),
                      _full((3, C)), _full((1, C))],
            out_specs=out_specs,
        )(x, aggp.reshape(2, 5, NPAD, W), g2[i], b2[i].reshape(1, C),
          Wa1[i], Wa2[i], gnx, bnx.reshape(1, C))
        if last:
            hfin = res[0]
        else:
            x, h = res

    return jnp.transpose(hfin, (1, 0, 2))
